# baseline (speedup 1.0000x reference)
import functools

import jax
import jax.numpy as jnp
from jax import lax
from jax.experimental import pallas as pl
from jax.experimental.pallas import tpu as pltpu
from jax.experimental.pallas import tpu_sc as plsc

NUM_CORES = 2
NUM_SUBCORES = 16
NUM_WORKERS = NUM_CORES * NUM_SUBCORES
LANES = 16


def _emb_body(per_w, chunk, n_chunks, d,
              x_hbm, pos_hbm, table_hbm, out_hbm,
              idx_v, rows_v, pos_v, sem):
    wid = lax.axis_index("s") * NUM_CORES + lax.axis_index("c")
    wbase = wid * per_w
    vecs_per_row = d // LANES

    def chunk_body(g, carry):
        base = wbase + g * chunk
        pltpu.sync_copy(x_hbm.at[pl.ds(base, chunk)], idx_v)
        gather = pltpu.async_copy(table_hbm.at[idx_v], rows_v, sem)
        pltpu.sync_copy(pos_hbm.at[pl.ds(base, chunk)], pos_v)
        gather.wait()

        def add_body(i, c):
            for j in range(vecs_per_row):
                sl = pl.ds(j * LANES, LANES)
                pos_v[i, sl] = pos_v[i, sl] + rows_v[i, sl]
            return c

        lax.fori_loop(0, chunk, add_body, 0, unroll=4)
        pltpu.sync_copy(pos_v, out_hbm.at[pl.ds(base, chunk)])
        return carry

    lax.fori_loop(0, n_chunks, chunk_body, 0)


def kernel(x, positional_embedding, table):
    b, s = x.shape
    v, d = table.shape
    n = b * s
    per_w = n // NUM_WORKERS
    chunk = 512
    n_chunks = per_w // chunk
    assert per_w % chunk == 0 and n % NUM_WORKERS == 0 and d % LANES == 0

    xf = x.reshape(n).astype(jnp.int32)
    posf = positional_embedding.reshape(n, d)

    mesh = plsc.VectorSubcoreMesh(
        core_axis_name="c", subcore_axis_name="s",
        num_cores=NUM_CORES, num_subcores=NUM_SUBCORES)

    emb = functools.partial(
        pl.kernel,
        out_type=jax.ShapeDtypeStruct((n, d), jnp.float32),
        mesh=mesh,
        scratch_types=[
            pltpu.VMEM((chunk,), jnp.int32),
            pltpu.VMEM((chunk, d), jnp.float32),
            pltpu.VMEM((chunk, d), jnp.float32),
            pltpu.SemaphoreType.DMA,
        ],
        compiler_params=pltpu.CompilerParams(use_tc_tiling_on_sc=False),
    )(functools.partial(_emb_body, per_w, chunk, n_chunks, d))

    out = emb(xf, posf, table)
    return out.reshape(b, s, d)


# speedup vs baseline: 1.3066x; 1.3066x over previous
import functools

import jax
import jax.numpy as jnp
from jax import lax
from jax.experimental import pallas as pl
from jax.experimental.pallas import tpu as pltpu
from jax.experimental.pallas import tpu_sc as plsc

NUM_CORES = 2
NUM_SUBCORES = 16
NUM_WORKERS = NUM_CORES * NUM_SUBCORES


def _emb_body(per_w, chunk, n_chunks, d,
              x_hbm, pos_hbm, table_hbm, out_hbm,
              idx_v, acc_v, pos_sem, g_sem, o_sem):
    wid = lax.axis_index("s") * NUM_CORES + lax.axis_index("c")
    wbase = wid * per_w

    def load_pos(g, b):
        base = wbase + g * chunk
        pltpu.sync_copy(x_hbm.at[pl.ds(base, chunk)], idx_v.at[b])
        pltpu.async_copy(pos_hbm.at[pl.ds(base, chunk)], acc_v.at[b],
                         pos_sem.at[b])

    def wait_pos(b):
        pltpu.make_async_copy(pos_hbm.at[pl.ds(0, chunk)], acc_v.at[b],
                              pos_sem.at[b]).wait()

    def wait_out(b):
        pltpu.make_async_copy(acc_v.at[b], out_hbm.at[pl.ds(0, chunk)],
                              o_sem.at[b]).wait()

    load_pos(0, 0)

    def pair_body(p, carry):
        for b in range(2):
            g = 2 * p + b
            wait_pos(b)
            gather = pltpu.async_copy(table_hbm.at[idx_v.at[b]], acc_v.at[b],
                                      g_sem.at[b], add=True)
            nb = 1 - b
            if b == 0:

                @pl.when(g >= 1)
                def _():
                    wait_out(nb)

                load_pos(g + 1, nb)
            else:

                @pl.when(g + 1 < n_chunks)
                def _():
                    wait_out(nb)
                    load_pos(g + 1, nb)

            gather.wait()
            base = wbase + g * chunk
            pltpu.async_copy(acc_v.at[b], out_hbm.at[pl.ds(base, chunk)],
                             o_sem.at[b])
        return carry

    lax.fori_loop(0, n_chunks // 2, pair_body, 0)
    wait_out(0)
    wait_out(1)


def kernel(x, positional_embedding, table):
    b, s = x.shape
    v, d = table.shape
    n = b * s
    per_w = n // NUM_WORKERS
    chunk = 800
    n_chunks = per_w // chunk
    assert per_w % chunk == 0 and n % NUM_WORKERS == 0
    assert n_chunks % 2 == 0 and chunk % 8 == 0

    xf = x.reshape(n).astype(jnp.int32)
    posf = positional_embedding.reshape(n, d)

    mesh = plsc.VectorSubcoreMesh(
        core_axis_name="c", subcore_axis_name="s",
        num_cores=NUM_CORES, num_subcores=NUM_SUBCORES)

    emb = functools.partial(
        pl.kernel,
        out_type=jax.ShapeDtypeStruct((n, d), jnp.float32),
        mesh=mesh,
        scratch_types=[
            pltpu.VMEM((2, chunk), jnp.int32),
            pltpu.VMEM((2, chunk, d), jnp.float32),
            pltpu.SemaphoreType.DMA((2,)),
            pltpu.SemaphoreType.DMA((2,)),
            pltpu.SemaphoreType.DMA((2,)),
        ],
        compiler_params=pltpu.CompilerParams(use_tc_tiling_on_sc=False),
    )(functools.partial(_emb_body, per_w, chunk, n_chunks, d))

    out = emb(xf, posf, table)
    return out.reshape(b, s, d)
